# baseline (speedup 1.0000x reference)
import functools

import jax
import jax.numpy as jnp
from jax import lax
from jax.experimental import pallas as pl
from jax.experimental.pallas import tpu as pltpu
from jax.experimental.pallas import tpu_sc as plsc

B, C, H, W = 16, 384, 64, 64
ROW = H * W
NROWS = B * C
NW = 32
ROWS_PER_W = NROWS // NW


def _body(in_hbm, out_hbm):
    wid = lax.axis_index("s") * 2 + lax.axis_index("c")
    b = wid // 2
    c0 = (wid % 2) * ROWS_PER_W
    base = b * C

    def step(i, carry):
        c = c0 + i
        pltpu.sync_copy(in_hbm.at[base + (C - 1 - c)], out_hbm.at[base + c])
        return carry

    lax.fori_loop(0, ROWS_PER_W, step, 0)


@jax.jit
def kernel(input):
    flat = input.reshape(NROWS, ROW)
    mesh = plsc.VectorSubcoreMesh(core_axis_name="c", subcore_axis_name="s")
    out = pl.kernel(
        _body,
        out_type=jax.ShapeDtypeStruct((NROWS, ROW), jnp.float32),
        mesh=mesh,
    )(flat)
    return out.reshape(B, C, H, W)


# speedup vs baseline: 1.0011x; 1.0011x over previous
import functools

import jax
import jax.numpy as jnp
from jax import lax
from jax.experimental import pallas as pl
from jax.experimental.pallas import tpu as pltpu
from jax.experimental.pallas import tpu_sc as plsc

B, C, H, W = 16, 384, 64, 64
ROW = H * W
NROWS = B * C
NW = 32
ROWS_PER_W = NROWS // NW


def _body(in_hbm, out_hbm, sem):
    wid = lax.axis_index("s") * 2 + lax.axis_index("c")
    b = wid // 2
    c0 = (wid % 2) * ROWS_PER_W
    base = b * C

    def fire(i, carry):
        c = c0 + i
        pltpu.make_async_copy(
            in_hbm.at[base + (C - 1 - c)], out_hbm.at[base + c], sem
        ).start()
        return carry

    lax.fori_loop(0, ROWS_PER_W, fire, 0)

    def drain(i, carry):
        c = c0 + i
        pltpu.make_async_copy(
            in_hbm.at[base + (C - 1 - c)], out_hbm.at[base + c], sem
        ).wait()
        return carry

    lax.fori_loop(0, ROWS_PER_W, drain, 0)


@jax.jit
def kernel(input):
    flat = input.reshape(NROWS, ROW)
    mesh = plsc.VectorSubcoreMesh(core_axis_name="c", subcore_axis_name="s")
    out = pl.kernel(
        _body,
        out_type=jax.ShapeDtypeStruct((NROWS, ROW), jnp.float32),
        mesh=mesh,
        scratch_types=[pltpu.SemaphoreType.DMA],
    )(flat)
    return out.reshape(B, C, H, W)


# speedup vs baseline: 6.0852x; 6.0784x over previous
import functools

import jax
import jax.numpy as jnp
from jax import lax
from jax.experimental import pallas as pl
from jax.experimental.pallas import tpu as pltpu
from jax.experimental.pallas import tpu_sc as plsc

B, C, H, W = 16, 384, 64, 64
ROW = H * W
NROWS = B * C
NW = 32
ROWS_PER_W = NROWS // NW


CK = 8
NCHUNK = ROWS_PER_W // CK


def _body(in_hbm, out_hbm, buf, sem_ld, sem_st):
    wid = lax.axis_index("s") * 2 + lax.axis_index("c")
    b = wid // 2
    c0 = (wid % 2) * ROWS_PER_W
    base = b * C

    def fire_loads(j):
        for r in range(CK):
            pltpu.make_async_copy(
                in_hbm.at[base + (C - 1) - c0 - (j * CK + r)],
                buf.at[j % 2, r],
                sem_ld,
            ).start()

    def wait_loads(j):
        for r in range(CK):
            pltpu.make_async_copy(
                in_hbm.at[base], buf.at[j % 2, r], sem_ld
            ).wait()

    def fire_store(j):
        pltpu.make_async_copy(
            buf.at[j % 2], out_hbm.at[pl.ds(base + c0 + j * CK, CK)], sem_st
        ).start()

    def wait_store(j):
        pltpu.make_async_copy(
            buf.at[j % 2], out_hbm.at[pl.ds(base + c0 + j * CK, CK)], sem_st
        ).wait()

    fire_loads(0)
    for j in range(NCHUNK):
        wait_loads(j)
        fire_store(j)
        if j >= 1:
            wait_store(j - 1)
        if j + 1 < NCHUNK:
            fire_loads(j + 1)
    wait_store(NCHUNK - 1)


@jax.jit
def kernel(input):
    flat = input.reshape(NROWS, ROW)
    mesh = plsc.VectorSubcoreMesh(core_axis_name="c", subcore_axis_name="s")
    out = pl.kernel(
        _body,
        out_type=jax.ShapeDtypeStruct((NROWS, ROW), jnp.float32),
        mesh=mesh,
        scratch_types=[
            pltpu.VMEM((2, CK, ROW), jnp.float32),
            pltpu.SemaphoreType.DMA,
            pltpu.SemaphoreType.DMA,
        ],
    )(flat)
    return out.reshape(B, C, H, W)


# speedup vs baseline: 6.1547x; 1.0114x over previous
import functools

import jax
import jax.numpy as jnp
from jax import lax
from jax.experimental import pallas as pl
from jax.experimental.pallas import tpu as pltpu
from jax.experimental.pallas import tpu_sc as plsc

B, C, H, W = 16, 384, 64, 64
ROW = H * W
NROWS = B * C
NW = 32
ROWS_PER_W = NROWS // NW


CK = 8
NCHUNK = ROWS_PER_W // CK
NBUF = 3


def _body(in_hbm, out_hbm, buf, sem_ld, sem_st):
    wid = lax.axis_index("s") * 2 + lax.axis_index("c")
    b = wid // 2
    c0 = (wid % 2) * ROWS_PER_W
    base = b * C

    def fire_loads(j):
        for r in range(CK):
            pltpu.make_async_copy(
                in_hbm.at[base + (C - 1) - c0 - (j * CK + r)],
                buf.at[j % NBUF, r],
                sem_ld,
            ).start()

    def wait_loads(j):
        for r in range(CK):
            pltpu.make_async_copy(
                in_hbm.at[base], buf.at[j % NBUF, r], sem_ld
            ).wait()

    def fire_store(j):
        pltpu.make_async_copy(
            buf.at[j % NBUF], out_hbm.at[pl.ds(base + c0 + j * CK, CK)], sem_st
        ).start()

    def wait_store(j):
        pltpu.make_async_copy(
            buf.at[j % NBUF], out_hbm.at[pl.ds(base + c0 + j * CK, CK)], sem_st
        ).wait()

    for j in range(NBUF - 1):
        fire_loads(j)
    for j in range(NCHUNK):
        wait_loads(j)
        fire_store(j)
        nxt = j + NBUF - 1
        if nxt < NCHUNK:
            if j >= 1:
                wait_store(j - 1)
            fire_loads(nxt)
    for j in range(NCHUNK - NBUF, NCHUNK):
        if j >= 0:
            wait_store(j)


@jax.jit
def kernel(input):
    flat = input.reshape(NROWS, ROW)
    mesh = plsc.VectorSubcoreMesh(core_axis_name="c", subcore_axis_name="s")
    out = pl.kernel(
        _body,
        out_type=jax.ShapeDtypeStruct((NROWS, ROW), jnp.float32),
        mesh=mesh,
        scratch_types=[
            pltpu.VMEM((NBUF, CK, ROW), jnp.float32),
            pltpu.SemaphoreType.DMA,
            pltpu.SemaphoreType.DMA,
        ],
    )(flat)
    return out.reshape(B, C, H, W)
